# baseline (speedup 1.0000x reference)
import functools

import jax
import jax.numpy as jnp
from jax import lax
from jax.experimental import pallas as pl
from jax.experimental.pallas import tpu as pltpu
from jax.experimental.pallas import tpu_sc as plsc

NC = 2
NS = 16
CH = 120


def _cdiv(a, b):
    return (a + b - 1) // b


def _deg_body(nb, ci, dst_hbm, out_hbm, dst_v, hist_v, sem):
    c = lax.axis_index("c")
    s = lax.axis_index("s")
    pltpu.async_copy(dst_hbm.at[c, s], dst_v, sem).wait()

    def zero(i, _):
        hist_v[pl.ds(i * 16, 16)] = jnp.zeros((16,), jnp.float32)
        return _

    lax.fori_loop(0, nb // 16, zero, None)
    ones = jnp.ones((16,), jnp.float32)

    def acc(i, _):
        plsc.addupdate_scatter(hist_v, [dst_v[i]], ones)
        return _

    lax.fori_loop(0, ci, acc, None)
    pltpu.sync_copy(hist_v, out_hbm.at[c, s])


def _degree_hist(dst, n, nb):
    e = dst.shape[0]
    ci = _cdiv(e, NC * NS * 16)
    e_pad = NC * NS * 16 * ci
    if e_pad != e:
        dst = jnp.concatenate(
            [dst, jnp.full((e_pad - e,), n, jnp.int32)])
    dst_r = dst.reshape(NC, NS, ci, 16)
    k = functools.partial(
        pl.kernel,
        out_type=jax.ShapeDtypeStruct((NC, NS, nb), jnp.float32),
        mesh=plsc.VectorSubcoreMesh(core_axis_name="c", subcore_axis_name="s",
                                    num_cores=NC, num_subcores=NS),
        scratch_types=[
            pltpu.VMEM((ci, 16), jnp.int32),
            pltpu.VMEM((nb,), jnp.float32),
            pltpu.SemaphoreType.DMA,
        ],
        compiler_params=pltpu.CompilerParams(needs_layout_passes=False),
    )(functools.partial(_deg_body, nb, ci))
    return k(dst_r).reshape(NC * NS, nb)


_K = 3


def _agg_body(cc0, cc1, n_out, ht_hbm, srcf_hbm, dstf_hbm,
              zeros_hbm, out_hbm,
              s0, s1, s2, d0, d1, d2, r0, r1, r2, acc_sh,
              issem, idsem, gsem, ssem):
    sidx = (s0, s1, s2)
    didx = (d0, d1, d2)
    rows = (r0, r1, r2)
    c = lax.axis_index("c")
    s = lax.axis_index("s")
    base = jnp.where(c == 0, s * cc0, NS * cc0 + s * cc1)
    nck = jnp.where(c == 0, cc0, cc1)

    def ld_s(j, b):
        return pltpu.make_async_copy(
            srcf_hbm.at[pl.ds((base + j) * CH, CH)], sidx[b], issem.at[b])

    def ld_d(j, b):
        return pltpu.make_async_copy(
            dstf_hbm.at[pl.ds((base + j) * CH, CH)], didx[b], idsem.at[b])

    def gather(b):
        return pltpu.make_async_copy(
            ht_hbm.at[sidx[b]], rows[b], gsem.at[b])

    def scatter(b):
        return pltpu.make_async_copy(
            rows[b], acc_sh.at[didx[b]], ssem.at[b])

    rpt = n_out // NS
    pltpu.sync_copy(zeros_hbm.at[pl.ds(s * rpt, rpt)],
                    acc_sh.at[pl.ds(s * rpt, rpt)])
    plsc.subcore_barrier()

    for b in range(_K):
        ld_s(b, b).start()
        ld_d(b, b).start()
    for b in range(_K):
        ld_s(b, b).wait()
        gather(b).start()

    def group(g, _):
        j0 = g * _K
        for b in range(_K):
            j = j0 + b
            jn = j + _K
            gather(b).wait()

            @pl.when(jn < nck)
            def _():
                ld_s(jn, b).start()
            ld_d(j, b).wait()
            scatter(b).start(add=True)
            scatter(b).wait()

            @pl.when(jn < nck)
            def _():
                ld_d(jn, b).start()
                ld_s(jn, b).wait()
                gather(b).start()
        return _

    lax.fori_loop(0, nck // _K, group, None)
    plsc.subcore_barrier()
    pltpu.sync_copy(acc_sh.at[pl.ds(s * rpt, rpt)],
                    out_hbm.at[c, pl.ds(s * rpt, rpt)])


def _aggregate(ht, src_flat, dst_flat, zeros_nd, cc0, cc1):
    n, d = ht.shape
    n1 = zeros_nd.shape[0]
    idx_t = pltpu.VMEM((CH,), jnp.int32)
    row_t = pltpu.VMEM((CH, d), jnp.float32)
    k = functools.partial(
        pl.kernel,
        out_type=jax.ShapeDtypeStruct((NC, n1, d), jnp.float32),
        mesh=plsc.VectorSubcoreMesh(core_axis_name="c", subcore_axis_name="s",
                                    num_cores=NC, num_subcores=NS),
        scratch_types=[
            idx_t, idx_t, idx_t, idx_t, idx_t, idx_t,
            row_t, row_t, row_t,
            pltpu.VMEM_SHARED((n1, d), jnp.float32),
            pltpu.SemaphoreType.DMA((_K,)),
            pltpu.SemaphoreType.DMA((_K,)),
            pltpu.SemaphoreType.DMA((_K,)),
            pltpu.SemaphoreType.DMA((_K,)),
        ],
    )(functools.partial(_agg_body, cc0, cc1, n1))
    return k(ht, src_flat, dst_flat, zeros_nd)


def _dinv_kernel(hist_ref, out_ref):
    deg = jnp.sum(hist_ref[...], axis=0) + 1.0
    out_ref[...] = lax.rsqrt(deg)


def _compute_dinv(hist_parts, nb):
    rows = nb // 128
    h3 = hist_parts.reshape(NC * NS, rows, 128)
    out = pl.pallas_call(
        _dinv_kernel,
        out_shape=jax.ShapeDtypeStruct((rows, 128), jnp.float32),
    )(h3)
    return out.reshape(nb)


def _mm1_kernel(x_ref, w_ref, dinv_ref, out_ref):
    out_ref[...] = jnp.dot(x_ref[...], w_ref[...],
                           preferred_element_type=jnp.float32) * dinv_ref[...]


def _fused_kernel(p0_ref, p1_ref, htp_ref, dinv_ref, b_ref, w_ref, out_ref):
    h = (p0_ref[...] + p1_ref[...] + htp_ref[...]) * dinv_ref[...] + b_ref[...]
    h = jnp.maximum(h, 0.0)
    out_ref[...] = jnp.dot(h, w_ref[...],
                           preferred_element_type=jnp.float32) * dinv_ref[...]


def _final_kernel(dout, p0_ref, p1_ref, htp_ref, dinv_ref, b_ref, out_ref):
    s = (p0_ref[...] + p1_ref[...] + htp_ref[...])[:, :dout]
    out_ref[...] = s * dinv_ref[...] + b_ref[...]


_BN = 1000


def _row_spec(d):
    return pl.BlockSpec((_BN, d), lambda i: (i, 0))


def _full_spec(shape):
    return pl.BlockSpec(shape, lambda i: (0,) * len(shape))


def _mm1(x, w, dinv_col):
    n, din = x.shape
    dout = w.shape[1]
    return pl.pallas_call(
        _mm1_kernel,
        grid=(n // _BN,),
        in_specs=[_row_spec(din), _full_spec(w.shape),
                  pl.BlockSpec((_BN, 1), lambda i: (i, 0))],
        out_specs=_row_spec(dout),
        out_shape=jax.ShapeDtypeStruct((n, dout), jnp.float32),
    )(x, w, dinv_col)


def _fused(parts, htp, dinv_col, b, w):
    n, din = htp.shape
    dout = w.shape[1]
    return pl.pallas_call(
        _fused_kernel,
        grid=(n // _BN,),
        in_specs=[_row_spec(din), _row_spec(din), _row_spec(din),
                  pl.BlockSpec((_BN, 1), lambda i: (i, 0)),
                  _full_spec((1, din)), _full_spec(w.shape)],
        out_specs=_row_spec(dout),
        out_shape=jax.ShapeDtypeStruct((n, dout), jnp.float32),
    )(parts[0], parts[1], htp, dinv_col, b.reshape(1, din), w)


def _final(parts, htp, dinv_col, b):
    n, d = htp.shape
    dout = b.shape[0]
    return pl.pallas_call(
        functools.partial(_final_kernel, dout),
        grid=(n // _BN,),
        in_specs=[_row_spec(d), _row_spec(d), _row_spec(d),
                  pl.BlockSpec((_BN, 1), lambda i: (i, 0)),
                  _full_spec((1, dout))],
        out_specs=_row_spec(dout),
        out_shape=jax.ShapeDtypeStruct((n, dout), jnp.float32),
    )(parts[0], parts[1], htp, dinv_col, b.reshape(1, dout))


def kernel(x, edge_index, W1, b1, W2, b2, W3, b3):
    n, d_in = x.shape
    e = edge_index.shape[1]
    src = edge_index[0]
    dst = edge_index[1]

    nb = _cdiv(n + 1, 128) * 128
    n1 = _cdiv(n + 1, NS * 8) * NS * 8

    cc_tot = NC * _cdiv(e, NC * NS * CH * _K) * _K
    cc0 = (cc_tot // (2 * _K)) * _K + 21
    cc1 = cc_tot - cc0
    e_pad = NS * cc_tot * CH
    src_r = jnp.concatenate([src, jnp.zeros((e_pad - e,), jnp.int32)])
    dst_r = jnp.concatenate([dst, jnp.full((e_pad - e,), n, jnp.int32)])

    zeros_wide = jnp.zeros((n1, d_in), jnp.float32)

    hist_parts = _degree_hist(dst, n, nb)
    dinv = _compute_dinv(hist_parts, nb)
    dinv_col = dinv[:n].reshape(n, 1)

    ht1 = _mm1(x, W1, dinv_col)
    p1 = _aggregate(ht1, src_r, dst_r, zeros_wide, cc0, cc1)
    ht2 = _fused(p1, ht1, dinv_col, b1, W2)
    p2 = _aggregate(ht2, src_r, dst_r, zeros_wide, cc0, cc1)
    w3_pad = jnp.concatenate(
        [W3, jnp.zeros((W3.shape[0], 128 - W3.shape[1]), jnp.float32)], axis=1)
    ht3 = _fused(p2, ht2, dinv_col, b2, w3_pad)
    p3 = _aggregate(ht3, src_r, dst_r, zeros_wide, cc0, cc1)
    mu = _final(p3, ht3, dinv_col, b3)
    return (mu, mu)


# speedup vs baseline: 1.0382x; 1.0382x over previous
import functools

import jax
import jax.numpy as jnp
from jax import lax
from jax.experimental import pallas as pl
from jax.experimental.pallas import tpu as pltpu
from jax.experimental.pallas import tpu_sc as plsc

NC = 2
NS = 16
CH = 120


def _cdiv(a, b):
    return (a + b - 1) // b


def _deg_body(nb, ci, dst_hbm, out_hbm, dst_v, hist_v, sem):
    c = lax.axis_index("c")
    s = lax.axis_index("s")
    pltpu.async_copy(dst_hbm.at[c, s], dst_v, sem).wait()

    def zero(i, _):
        hist_v[pl.ds(i * 16, 16)] = jnp.zeros((16,), jnp.float32)
        return _

    lax.fori_loop(0, nb // 16, zero, None)
    ones = jnp.ones((16,), jnp.float32)

    def acc(i, _):
        plsc.addupdate_scatter(hist_v, [dst_v[i]], ones)
        return _

    lax.fori_loop(0, ci, acc, None)
    pltpu.sync_copy(hist_v, out_hbm.at[c, s])


def _degree_hist(dst, n, nb):
    e = dst.shape[0]
    ci = _cdiv(e, NC * NS * 16)
    e_pad = NC * NS * 16 * ci
    if e_pad != e:
        dst = jnp.concatenate(
            [dst, jnp.full((e_pad - e,), n, jnp.int32)])
    dst_r = dst.reshape(NC, NS, ci, 16)
    k = functools.partial(
        pl.kernel,
        out_type=jax.ShapeDtypeStruct((NC, NS, nb), jnp.float32),
        mesh=plsc.VectorSubcoreMesh(core_axis_name="c", subcore_axis_name="s",
                                    num_cores=NC, num_subcores=NS),
        scratch_types=[
            pltpu.VMEM((ci, 16), jnp.int32),
            pltpu.VMEM((nb,), jnp.float32),
            pltpu.SemaphoreType.DMA,
        ],
        compiler_params=pltpu.CompilerParams(needs_layout_passes=False),
    )(functools.partial(_deg_body, nb, ci))
    return k(dst_r).reshape(NC * NS, nb)


_K = 3


def _agg_body(cc0, cc1, n_out, ht_hbm, srcf_hbm, dstf_hbm,
              zeros_hbm, out_hbm,
              s0, s1, s2, d0, d1, d2, r0, r1, r2, acc_sh,
              issem, idsem, gsem, ssem):
    sidx = (s0, s1, s2)
    didx = (d0, d1, d2)
    rows = (r0, r1, r2)
    c = lax.axis_index("c")
    s = lax.axis_index("s")
    base = jnp.where(c == 0, s * cc0, NS * cc0 + s * cc1)
    nck = jnp.where(c == 0, cc0, cc1)

    def ld_s(j, b):
        return pltpu.make_async_copy(
            srcf_hbm.at[pl.ds((base + j) * CH, CH)], sidx[b], issem.at[b])

    def ld_d(j, b):
        return pltpu.make_async_copy(
            dstf_hbm.at[pl.ds((base + j) * CH, CH)], didx[b], idsem.at[b])

    def gather(b):
        return pltpu.make_async_copy(
            ht_hbm.at[sidx[b]], rows[b], gsem.at[b])

    def scatter(b):
        return pltpu.make_async_copy(
            rows[b], acc_sh.at[didx[b]], ssem.at[b])

    rpt = n_out // NS
    pltpu.sync_copy(zeros_hbm.at[pl.ds(s * rpt, rpt)],
                    acc_sh.at[pl.ds(s * rpt, rpt)])
    plsc.subcore_barrier()

    for b in range(_K):
        ld_s(b, b).start()
        ld_d(b, b).start()
    for b in range(_K):
        ld_s(b, b).wait()
        gather(b).start()

    def group(g, _):
        j0 = g * _K
        for b in range(_K):
            j = j0 + b
            jn = j + _K
            gather(b).wait()

            @pl.when(jn < nck)
            def _():
                ld_s(jn, b).start()
            ld_d(j, b).wait()
            scatter(b).start(add=True)
            scatter(b).wait()

            @pl.when(jn < nck)
            def _():
                ld_d(jn, b).start()
                ld_s(jn, b).wait()
                gather(b).start()
        return _

    lax.fori_loop(0, nck // _K, group, None)
    plsc.subcore_barrier()
    pltpu.sync_copy(acc_sh.at[pl.ds(s * rpt, rpt)],
                    out_hbm.at[c, pl.ds(s * rpt, rpt)])


def _aggregate(ht, src_flat, dst_flat, zeros_nd, cc0, cc1):
    n, d = ht.shape
    n1 = zeros_nd.shape[0]
    idx_t = pltpu.VMEM((CH,), jnp.int32)
    row_t = pltpu.VMEM((CH, d), jnp.float32)
    k = functools.partial(
        pl.kernel,
        out_type=jax.ShapeDtypeStruct((NC, n1, d), jnp.float32),
        mesh=plsc.VectorSubcoreMesh(core_axis_name="c", subcore_axis_name="s",
                                    num_cores=NC, num_subcores=NS),
        scratch_types=[
            idx_t, idx_t, idx_t, idx_t, idx_t, idx_t,
            row_t, row_t, row_t,
            pltpu.VMEM_SHARED((n1, d), jnp.float32),
            pltpu.SemaphoreType.DMA((_K,)),
            pltpu.SemaphoreType.DMA((_K,)),
            pltpu.SemaphoreType.DMA((_K,)),
            pltpu.SemaphoreType.DMA((_K,)),
        ],
    )(functools.partial(_agg_body, cc0, cc1, n1))
    return k(ht, src_flat, dst_flat, zeros_nd)


def _dinv_kernel(hist_ref, out_ref):
    deg = jnp.sum(hist_ref[...], axis=0) + 1.0
    out_ref[...] = lax.rsqrt(deg)


def _compute_dinv(hist_parts, nb):
    rows = nb // 128
    h3 = hist_parts.reshape(NC * NS, rows, 128)
    out = pl.pallas_call(
        _dinv_kernel,
        out_shape=jax.ShapeDtypeStruct((rows, 128), jnp.float32),
    )(h3)
    return out.reshape(nb)


def _mm1_kernel(x_ref, w_ref, dinv_ref, out_ref):
    out_ref[...] = jnp.dot(x_ref[...], w_ref[...],
                           preferred_element_type=jnp.float32) * dinv_ref[...]


def _fused_kernel(p0_ref, p1_ref, htp_ref, dinv_ref, b_ref, w_ref, out_ref):
    h = (p0_ref[...] + p1_ref[...] + htp_ref[...]) * dinv_ref[...] + b_ref[...]
    h = jnp.maximum(h, 0.0)
    out_ref[...] = jnp.dot(h, w_ref[...],
                           preferred_element_type=jnp.float32) * dinv_ref[...]


def _final_kernel(dout, p0_ref, p1_ref, htp_ref, dinv_ref, b_ref, out_ref):
    s = (p0_ref[...] + p1_ref[...] + htp_ref[...])[:, :dout]
    out_ref[...] = s * dinv_ref[...] + b_ref[...]


_BN = 1000


def _row_spec(d):
    return pl.BlockSpec((_BN, d), lambda i: (i, 0))


def _full_spec(shape):
    return pl.BlockSpec(shape, lambda i: (0,) * len(shape))


def _mm1(x, w, dinv_col):
    n, din = x.shape
    dout = w.shape[1]
    return pl.pallas_call(
        _mm1_kernel,
        grid=(n // _BN,),
        in_specs=[_row_spec(din), _full_spec(w.shape),
                  pl.BlockSpec((_BN, 1), lambda i: (i, 0))],
        out_specs=_row_spec(dout),
        out_shape=jax.ShapeDtypeStruct((n, dout), jnp.float32),
    )(x, w, dinv_col)


def _fused(parts, htp, dinv_col, b, w):
    n, din = htp.shape
    dout = w.shape[1]
    return pl.pallas_call(
        _fused_kernel,
        grid=(n // _BN,),
        in_specs=[_row_spec(din), _row_spec(din), _row_spec(din),
                  pl.BlockSpec((_BN, 1), lambda i: (i, 0)),
                  _full_spec((1, din)), _full_spec(w.shape)],
        out_specs=_row_spec(dout),
        out_shape=jax.ShapeDtypeStruct((n, dout), jnp.float32),
    )(parts[0], parts[1], htp, dinv_col, b.reshape(1, din), w)


def _final(parts, htp, dinv_col, b):
    n, d = htp.shape
    dout = b.shape[0]
    return pl.pallas_call(
        functools.partial(_final_kernel, dout),
        grid=(n // _BN,),
        in_specs=[_row_spec(d), _row_spec(d), _row_spec(d),
                  pl.BlockSpec((_BN, 1), lambda i: (i, 0)),
                  _full_spec((1, dout))],
        out_specs=_row_spec(dout),
        out_shape=jax.ShapeDtypeStruct((n, dout), jnp.float32),
    )(parts[0], parts[1], htp, dinv_col, b.reshape(1, dout))


def kernel(x, edge_index, W1, b1, W2, b2, W3, b3):
    n, d_in = x.shape
    e = edge_index.shape[1]
    src = edge_index[0]
    dst = edge_index[1]

    nb = _cdiv(n + 1, 128) * 128
    n1 = _cdiv(n + 1, NS * 8) * NS * 8

    cc_tot = NC * _cdiv(e, NC * NS * CH * _K) * _K
    cc0 = (cc_tot // (2 * _K)) * _K + 33
    cc1 = cc_tot - cc0
    e_pad = NS * cc_tot * CH
    src_r = jnp.concatenate([src, jnp.zeros((e_pad - e,), jnp.int32)])
    dst_r = jnp.concatenate([dst, jnp.full((e_pad - e,), n, jnp.int32)])

    zeros_wide = jnp.zeros((n1, d_in), jnp.float32)

    hist_parts = _degree_hist(dst, n, nb)
    dinv = _compute_dinv(hist_parts, nb)
    dinv_col = dinv[:n].reshape(n, 1)

    ht1 = _mm1(x, W1, dinv_col)
    p1 = _aggregate(ht1, src_r, dst_r, zeros_wide, cc0, cc1)
    ht2 = _fused(p1, ht1, dinv_col, b1, W2)
    p2 = _aggregate(ht2, src_r, dst_r, zeros_wide, cc0, cc1)
    w3_pad = jnp.concatenate(
        [W3, jnp.zeros((W3.shape[0], 128 - W3.shape[1]), jnp.float32)], axis=1)
    ht3 = _fused(p2, ht2, dinv_col, b2, w3_pad)
    p3 = _aggregate(ht3, src_r, dst_r, zeros_wide, cc0, cc1)
    mu = _final(p3, ht3, dinv_col, b3)
    return (mu, mu)


# speedup vs baseline: 1.0707x; 1.0312x over previous
import functools

import jax
import jax.numpy as jnp
from jax import lax
from jax.experimental import pallas as pl
from jax.experimental.pallas import tpu as pltpu
from jax.experimental.pallas import tpu_sc as plsc

NC = 2
NS = 16
CH = 120


def _cdiv(a, b):
    return (a + b - 1) // b


def _deg_body(nb, ci, dst_hbm, out_hbm, dst_v, hist_v, sem):
    c = lax.axis_index("c")
    s = lax.axis_index("s")
    pltpu.async_copy(dst_hbm.at[c, s], dst_v, sem).wait()

    def zero(i, _):
        hist_v[pl.ds(i * 16, 16)] = jnp.zeros((16,), jnp.float32)
        return _

    lax.fori_loop(0, nb // 16, zero, None)
    ones = jnp.ones((16,), jnp.float32)

    def acc(i, _):
        plsc.addupdate_scatter(hist_v, [dst_v[i]], ones)
        return _

    lax.fori_loop(0, ci, acc, None)
    pltpu.sync_copy(hist_v, out_hbm.at[c, s])


def _degree_hist(dst, n, nb):
    e = dst.shape[0]
    ci = _cdiv(e, NC * NS * 16)
    e_pad = NC * NS * 16 * ci
    if e_pad != e:
        dst = jnp.concatenate(
            [dst, jnp.full((e_pad - e,), n, jnp.int32)])
    dst_r = dst.reshape(NC, NS, ci, 16)
    k = functools.partial(
        pl.kernel,
        out_type=jax.ShapeDtypeStruct((NC, NS, nb), jnp.float32),
        mesh=plsc.VectorSubcoreMesh(core_axis_name="c", subcore_axis_name="s",
                                    num_cores=NC, num_subcores=NS),
        scratch_types=[
            pltpu.VMEM((ci, 16), jnp.int32),
            pltpu.VMEM((nb,), jnp.float32),
            pltpu.SemaphoreType.DMA,
        ],
        compiler_params=pltpu.CompilerParams(needs_layout_passes=False),
    )(functools.partial(_deg_body, nb, ci))
    return k(dst_r).reshape(NC * NS, nb)


_K = 3


def _agg_body(cc0, cc1, n_out, ht_hbm, srcf_hbm, dstf_hbm,
              zeros_hbm, out_hbm,
              s0, s1, s2, d0, d1, d2, r0, r1, r2, acc_sh,
              issem, idsem, gsem, ssem):
    sidx = (s0, s1, s2)
    didx = (d0, d1, d2)
    rows = (r0, r1, r2)
    c = lax.axis_index("c")
    s = lax.axis_index("s")
    base = jnp.where(c == 0, s * cc0, NS * cc0 + s * cc1)
    nck = jnp.where(c == 0, cc0, cc1)

    def ld_s(j, b):
        return pltpu.make_async_copy(
            srcf_hbm.at[pl.ds((base + j) * CH, CH)], sidx[b], issem.at[b])

    def ld_d(j, b):
        return pltpu.make_async_copy(
            dstf_hbm.at[pl.ds((base + j) * CH, CH)], didx[b], idsem.at[b])

    def gather(b):
        return pltpu.make_async_copy(
            ht_hbm.at[sidx[b]], rows[b], gsem.at[b])

    def scatter(b):
        return pltpu.make_async_copy(
            rows[b], acc_sh.at[didx[b]], ssem.at[b])

    rpt = n_out // NS
    pltpu.sync_copy(zeros_hbm.at[pl.ds(s * rpt, rpt)],
                    acc_sh.at[pl.ds(s * rpt, rpt)])
    plsc.subcore_barrier()

    for b in range(_K):
        ld_s(b, b).start()
        ld_d(b, b).start()
    for b in range(_K):
        ld_s(b, b).wait()
        gather(b).start()

    def group(g, _):
        j0 = g * _K
        for b in range(_K):
            j = j0 + b
            jn = j + _K
            gather(b).wait()

            @pl.when(jn < nck)
            def _():
                ld_s(jn, b).start()
            ld_d(j, b).wait()
            scatter(b).start(add=True)
            scatter(b).wait()

            @pl.when(jn < nck)
            def _():
                ld_d(jn, b).start()
                ld_s(jn, b).wait()
                gather(b).start()
        return _

    lax.fori_loop(0, nck // _K, group, None)
    plsc.subcore_barrier()
    pltpu.sync_copy(acc_sh.at[pl.ds(s * rpt, rpt)],
                    out_hbm.at[c, pl.ds(s * rpt, rpt)])


def _aggregate(ht, src_flat, dst_flat, zeros_nd, cc0, cc1):
    n, d = ht.shape
    n1 = zeros_nd.shape[0]
    idx_t = pltpu.VMEM((CH,), jnp.int32)
    row_t = pltpu.VMEM((CH, d), jnp.float32)
    k = functools.partial(
        pl.kernel,
        out_type=jax.ShapeDtypeStruct((NC, n1, d), jnp.float32),
        mesh=plsc.VectorSubcoreMesh(core_axis_name="c", subcore_axis_name="s",
                                    num_cores=NC, num_subcores=NS),
        scratch_types=[
            idx_t, idx_t, idx_t, idx_t, idx_t, idx_t,
            row_t, row_t, row_t,
            pltpu.VMEM_SHARED((n1, d), jnp.float32),
            pltpu.SemaphoreType.DMA((_K,)),
            pltpu.SemaphoreType.DMA((_K,)),
            pltpu.SemaphoreType.DMA((_K,)),
            pltpu.SemaphoreType.DMA((_K,)),
        ],
    )(functools.partial(_agg_body, cc0, cc1, n1))
    return k(ht, src_flat, dst_flat, zeros_nd)


def _dinv_kernel(hist_ref, out_ref):
    deg = jnp.sum(hist_ref[...], axis=0) + 1.0
    out_ref[...] = lax.rsqrt(deg)


def _compute_dinv(hist_parts, nb):
    rows = nb // 128
    h3 = hist_parts.reshape(NC * NS, rows, 128)
    out = pl.pallas_call(
        _dinv_kernel,
        out_shape=jax.ShapeDtypeStruct((rows, 128), jnp.float32),
    )(h3)
    return out.reshape(nb)


def _mm1_kernel(x_ref, w_ref, dinv_ref, out_ref):
    out_ref[...] = jnp.dot(x_ref[...], w_ref[...],
                           preferred_element_type=jnp.float32) * dinv_ref[...]


def _fused_kernel(p0_ref, p1_ref, htp_ref, dinv_ref, b_ref, w_ref, out_ref):
    h = (p0_ref[...] + p1_ref[...] + htp_ref[...]) * dinv_ref[...] + b_ref[...]
    h = jnp.maximum(h, 0.0)
    out_ref[...] = jnp.dot(h, w_ref[...],
                           preferred_element_type=jnp.float32) * dinv_ref[...]


def _final_kernel(dout, p0_ref, p1_ref, htp_ref, dinv_ref, b_ref, out_ref):
    s = (p0_ref[...] + p1_ref[...] + htp_ref[...])[:, :dout]
    out_ref[...] = s * dinv_ref[...] + b_ref[...]


_BN = 1000


def _row_spec(d):
    return pl.BlockSpec((_BN, d), lambda i: (i, 0))


def _full_spec(shape):
    return pl.BlockSpec(shape, lambda i: (0,) * len(shape))


def _mm1(x, w, dinv_col):
    n, din = x.shape
    dout = w.shape[1]
    return pl.pallas_call(
        _mm1_kernel,
        grid=(n // _BN,),
        in_specs=[_row_spec(din), _full_spec(w.shape),
                  pl.BlockSpec((_BN, 1), lambda i: (i, 0))],
        out_specs=_row_spec(dout),
        out_shape=jax.ShapeDtypeStruct((n, dout), jnp.float32),
    )(x, w, dinv_col)


def _fused(parts, htp, dinv_col, b, w):
    n, din = htp.shape
    dout = w.shape[1]
    return pl.pallas_call(
        _fused_kernel,
        grid=(n // _BN,),
        in_specs=[_row_spec(din), _row_spec(din), _row_spec(din),
                  pl.BlockSpec((_BN, 1), lambda i: (i, 0)),
                  _full_spec((1, din)), _full_spec(w.shape)],
        out_specs=_row_spec(dout),
        out_shape=jax.ShapeDtypeStruct((n, dout), jnp.float32),
    )(parts[0], parts[1], htp, dinv_col, b.reshape(1, din), w)


def _final(parts, htp, dinv_col, b):
    n, d = htp.shape
    dout = b.shape[0]
    return pl.pallas_call(
        functools.partial(_final_kernel, dout),
        grid=(n // _BN,),
        in_specs=[_row_spec(d), _row_spec(d), _row_spec(d),
                  pl.BlockSpec((_BN, 1), lambda i: (i, 0)),
                  _full_spec((1, dout))],
        out_specs=_row_spec(dout),
        out_shape=jax.ShapeDtypeStruct((n, dout), jnp.float32),
    )(parts[0], parts[1], htp, dinv_col, b.reshape(1, dout))


def kernel(x, edge_index, W1, b1, W2, b2, W3, b3):
    n, d_in = x.shape
    e = edge_index.shape[1]
    src = edge_index[0]
    dst = edge_index[1]

    nb = _cdiv(n + 1, 128) * 128
    n1 = _cdiv(n + 1, NS * 8) * NS * 8

    cc_tot = NC * _cdiv(e, NC * NS * CH * _K) * _K
    cc0 = (cc_tot // (2 * _K)) * _K + 45
    cc1 = cc_tot - cc0
    e_pad = NS * cc_tot * CH
    src_r = jnp.concatenate([src, jnp.zeros((e_pad - e,), jnp.int32)])
    dst_r = jnp.concatenate([dst, jnp.full((e_pad - e,), n, jnp.int32)])

    zeros_wide = jnp.zeros((n1, d_in), jnp.float32)

    hist_parts = _degree_hist(dst, n, nb)
    dinv = _compute_dinv(hist_parts, nb)
    dinv_col = dinv[:n].reshape(n, 1)

    ht1 = _mm1(x, W1, dinv_col)
    p1 = _aggregate(ht1, src_r, dst_r, zeros_wide, cc0, cc1)
    ht2 = _fused(p1, ht1, dinv_col, b1, W2)
    p2 = _aggregate(ht2, src_r, dst_r, zeros_wide, cc0, cc1)
    w3_pad = jnp.concatenate(
        [W3, jnp.zeros((W3.shape[0], 128 - W3.shape[1]), jnp.float32)], axis=1)
    ht3 = _fused(p2, ht2, dinv_col, b2, w3_pad)
    p3 = _aggregate(ht3, src_r, dst_r, zeros_wide, cc0, cc1)
    mu = _final(p3, ht3, dinv_col, b3)
    return (mu, mu)


# speedup vs baseline: 1.1108x; 1.0375x over previous
import functools

import jax
import jax.numpy as jnp
from jax import lax
from jax.experimental import pallas as pl
from jax.experimental.pallas import tpu as pltpu
from jax.experimental.pallas import tpu_sc as plsc

NC = 2
NS = 16
CH = 120


def _cdiv(a, b):
    return (a + b - 1) // b


def _deg_body(nb, ci, dst_hbm, out_hbm, dst_v, hist_v, sem):
    c = lax.axis_index("c")
    s = lax.axis_index("s")
    pltpu.async_copy(dst_hbm.at[c, s], dst_v, sem).wait()

    def zero(i, _):
        hist_v[pl.ds(i * 16, 16)] = jnp.zeros((16,), jnp.float32)
        return _

    lax.fori_loop(0, nb // 16, zero, None)
    ones = jnp.ones((16,), jnp.float32)

    def acc(i, _):
        plsc.addupdate_scatter(hist_v, [dst_v[i]], ones)
        return _

    lax.fori_loop(0, ci, acc, None)
    pltpu.sync_copy(hist_v, out_hbm.at[c, s])


def _degree_hist(dst, n, nb):
    e = dst.shape[0]
    ci = _cdiv(e, NC * NS * 16)
    e_pad = NC * NS * 16 * ci
    if e_pad != e:
        dst = jnp.concatenate(
            [dst, jnp.full((e_pad - e,), n, jnp.int32)])
    dst_r = dst.reshape(NC, NS, ci, 16)
    k = functools.partial(
        pl.kernel,
        out_type=jax.ShapeDtypeStruct((NC, NS, nb), jnp.float32),
        mesh=plsc.VectorSubcoreMesh(core_axis_name="c", subcore_axis_name="s",
                                    num_cores=NC, num_subcores=NS),
        scratch_types=[
            pltpu.VMEM((ci, 16), jnp.int32),
            pltpu.VMEM((nb,), jnp.float32),
            pltpu.SemaphoreType.DMA,
        ],
        compiler_params=pltpu.CompilerParams(needs_layout_passes=False),
    )(functools.partial(_deg_body, nb, ci))
    return k(dst_r).reshape(NC * NS, nb)


_K = 3


def _agg_body(cc0, cc1, n_out, ht_hbm, srcf_hbm, dstf_hbm,
              zeros_hbm, out_hbm,
              s0, s1, s2, d0, d1, d2, r0, r1, r2, acc_sh,
              issem, idsem, gsem, ssem):
    sidx = (s0, s1, s2)
    didx = (d0, d1, d2)
    rows = (r0, r1, r2)
    c = lax.axis_index("c")
    s = lax.axis_index("s")
    base = jnp.where(c == 0, s * cc0, NS * cc0 + s * cc1)
    nck = jnp.where(c == 0, cc0, cc1)

    def ld_s(j, b):
        return pltpu.make_async_copy(
            srcf_hbm.at[pl.ds((base + j) * CH, CH)], sidx[b], issem.at[b])

    def ld_d(j, b):
        return pltpu.make_async_copy(
            dstf_hbm.at[pl.ds((base + j) * CH, CH)], didx[b], idsem.at[b])

    def gather(b):
        return pltpu.make_async_copy(
            ht_hbm.at[sidx[b]], rows[b], gsem.at[b])

    def scatter(b):
        return pltpu.make_async_copy(
            rows[b], acc_sh.at[didx[b]], ssem.at[b])

    rpt = n_out // NS
    pltpu.sync_copy(zeros_hbm.at[pl.ds(s * rpt, rpt)],
                    acc_sh.at[pl.ds(s * rpt, rpt)])
    plsc.subcore_barrier()

    for b in range(_K):
        ld_s(b, b).start()
        ld_d(b, b).start()
    for b in range(_K):
        ld_s(b, b).wait()
        gather(b).start()

    def group(g, _):
        j0 = g * _K
        for b in range(_K):
            j = j0 + b
            jn = j + _K
            gather(b).wait()

            @pl.when(jn < nck)
            def _():
                ld_s(jn, b).start()
            ld_d(j, b).wait()
            scatter(b).start(add=True)
            scatter(b).wait()

            @pl.when(jn < nck)
            def _():
                ld_d(jn, b).start()
                ld_s(jn, b).wait()
                gather(b).start()
        return _

    lax.fori_loop(0, nck // _K, group, None)
    plsc.subcore_barrier()
    pltpu.sync_copy(acc_sh.at[pl.ds(s * rpt, rpt)],
                    out_hbm.at[c, pl.ds(s * rpt, rpt)])


def _aggregate(ht, src_flat, dst_flat, zeros_nd, cc0, cc1):
    n, d = ht.shape
    n1 = zeros_nd.shape[0]
    idx_t = pltpu.VMEM((CH,), jnp.int32)
    row_t = pltpu.VMEM((CH, d), jnp.float32)
    k = functools.partial(
        pl.kernel,
        out_type=jax.ShapeDtypeStruct((NC, n1, d), jnp.float32),
        mesh=plsc.VectorSubcoreMesh(core_axis_name="c", subcore_axis_name="s",
                                    num_cores=NC, num_subcores=NS),
        scratch_types=[
            idx_t, idx_t, idx_t, idx_t, idx_t, idx_t,
            row_t, row_t, row_t,
            pltpu.VMEM_SHARED((n1, d), jnp.float32),
            pltpu.SemaphoreType.DMA((_K,)),
            pltpu.SemaphoreType.DMA((_K,)),
            pltpu.SemaphoreType.DMA((_K,)),
            pltpu.SemaphoreType.DMA((_K,)),
        ],
    )(functools.partial(_agg_body, cc0, cc1, n1))
    return k(ht, src_flat, dst_flat, zeros_nd)


def _dinv_kernel(hist_ref, out_ref):
    deg = jnp.sum(hist_ref[...], axis=0) + 1.0
    out_ref[...] = lax.rsqrt(deg)


def _compute_dinv(hist_parts, nb):
    rows = nb // 128
    h3 = hist_parts.reshape(NC * NS, rows, 128)
    out = pl.pallas_call(
        _dinv_kernel,
        out_shape=jax.ShapeDtypeStruct((rows, 128), jnp.float32),
    )(h3)
    return out.reshape(nb)


def _mm1_kernel(x_ref, w_ref, dinv_ref, out_ref):
    out_ref[...] = jnp.dot(x_ref[...], w_ref[...],
                           preferred_element_type=jnp.float32) * dinv_ref[...]


def _fused_kernel(p0_ref, p1_ref, htp_ref, dinv_ref, b_ref, w_ref, out_ref):
    h = (p0_ref[...] + p1_ref[...] + htp_ref[...]) * dinv_ref[...] + b_ref[...]
    h = jnp.maximum(h, 0.0)
    out_ref[...] = jnp.dot(h, w_ref[...],
                           preferred_element_type=jnp.float32) * dinv_ref[...]


def _final_kernel(dout, p0_ref, p1_ref, htp_ref, dinv_ref, b_ref, out_ref):
    s = (p0_ref[...] + p1_ref[...] + htp_ref[...])[:, :dout]
    out_ref[...] = s * dinv_ref[...] + b_ref[...]


_BN = 1000


def _row_spec(d):
    return pl.BlockSpec((_BN, d), lambda i: (i, 0))


def _full_spec(shape):
    return pl.BlockSpec(shape, lambda i: (0,) * len(shape))


def _mm1(x, w, dinv_col):
    n, din = x.shape
    dout = w.shape[1]
    return pl.pallas_call(
        _mm1_kernel,
        grid=(n // _BN,),
        in_specs=[_row_spec(din), _full_spec(w.shape),
                  pl.BlockSpec((_BN, 1), lambda i: (i, 0))],
        out_specs=_row_spec(dout),
        out_shape=jax.ShapeDtypeStruct((n, dout), jnp.float32),
    )(x, w, dinv_col)


def _fused(parts, htp, dinv_col, b, w):
    n, din = htp.shape
    dout = w.shape[1]
    return pl.pallas_call(
        _fused_kernel,
        grid=(n // _BN,),
        in_specs=[_row_spec(din), _row_spec(din), _row_spec(din),
                  pl.BlockSpec((_BN, 1), lambda i: (i, 0)),
                  _full_spec((1, din)), _full_spec(w.shape)],
        out_specs=_row_spec(dout),
        out_shape=jax.ShapeDtypeStruct((n, dout), jnp.float32),
    )(parts[0], parts[1], htp, dinv_col, b.reshape(1, din), w)


def _final(parts, htp, dinv_col, b):
    n, d = htp.shape
    dout = b.shape[0]
    return pl.pallas_call(
        functools.partial(_final_kernel, dout),
        grid=(n // _BN,),
        in_specs=[_row_spec(d), _row_spec(d), _row_spec(d),
                  pl.BlockSpec((_BN, 1), lambda i: (i, 0)),
                  _full_spec((1, dout))],
        out_specs=_row_spec(dout),
        out_shape=jax.ShapeDtypeStruct((n, dout), jnp.float32),
    )(parts[0], parts[1], htp, dinv_col, b.reshape(1, dout))


def kernel(x, edge_index, W1, b1, W2, b2, W3, b3):
    n, d_in = x.shape
    e = edge_index.shape[1]
    src = edge_index[0]
    dst = edge_index[1]

    nb = _cdiv(n + 1, 128) * 128
    n1 = _cdiv(n + 1, NS * 8) * NS * 8

    cc_tot = NC * _cdiv(e, NC * NS * CH * _K) * _K
    cc0 = (cc_tot // (2 * _K)) * _K + 57
    cc1 = cc_tot - cc0
    e_pad = NS * cc_tot * CH
    src_r = jnp.concatenate([src, jnp.zeros((e_pad - e,), jnp.int32)])
    dst_r = jnp.concatenate([dst, jnp.full((e_pad - e,), n, jnp.int32)])

    zeros_wide = jnp.zeros((n1, d_in), jnp.float32)

    hist_parts = _degree_hist(dst, n, nb)
    dinv = _compute_dinv(hist_parts, nb)
    dinv_col = dinv[:n].reshape(n, 1)

    ht1 = _mm1(x, W1, dinv_col)
    p1 = _aggregate(ht1, src_r, dst_r, zeros_wide, cc0, cc1)
    ht2 = _fused(p1, ht1, dinv_col, b1, W2)
    p2 = _aggregate(ht2, src_r, dst_r, zeros_wide, cc0, cc1)
    w3_pad = jnp.concatenate(
        [W3, jnp.zeros((W3.shape[0], 128 - W3.shape[1]), jnp.float32)], axis=1)
    ht3 = _fused(p2, ht2, dinv_col, b2, w3_pad)
    p3 = _aggregate(ht3, src_r, dst_r, zeros_wide, cc0, cc1)
    mu = _final(p3, ht3, dinv_col, b3)
    return (mu, mu)


# speedup vs baseline: 1.1397x; 1.0260x over previous
import functools

import jax
import jax.numpy as jnp
from jax import lax
from jax.experimental import pallas as pl
from jax.experimental.pallas import tpu as pltpu
from jax.experimental.pallas import tpu_sc as plsc

NC = 2
NS = 16
CH = 120


def _cdiv(a, b):
    return (a + b - 1) // b


def _deg_body(nb, ci, dst_hbm, out_hbm, dst_v, hist_v, sem):
    c = lax.axis_index("c")
    s = lax.axis_index("s")
    pltpu.async_copy(dst_hbm.at[c, s], dst_v, sem).wait()

    def zero(i, _):
        hist_v[pl.ds(i * 16, 16)] = jnp.zeros((16,), jnp.float32)
        return _

    lax.fori_loop(0, nb // 16, zero, None)
    ones = jnp.ones((16,), jnp.float32)

    def acc(i, _):
        plsc.addupdate_scatter(hist_v, [dst_v[i]], ones)
        return _

    lax.fori_loop(0, ci, acc, None)
    pltpu.sync_copy(hist_v, out_hbm.at[c, s])


def _degree_hist(dst, n, nb):
    e = dst.shape[0]
    ci = _cdiv(e, NC * NS * 16)
    e_pad = NC * NS * 16 * ci
    if e_pad != e:
        dst = jnp.concatenate(
            [dst, jnp.full((e_pad - e,), n, jnp.int32)])
    dst_r = dst.reshape(NC, NS, ci, 16)
    k = functools.partial(
        pl.kernel,
        out_type=jax.ShapeDtypeStruct((NC, NS, nb), jnp.float32),
        mesh=plsc.VectorSubcoreMesh(core_axis_name="c", subcore_axis_name="s",
                                    num_cores=NC, num_subcores=NS),
        scratch_types=[
            pltpu.VMEM((ci, 16), jnp.int32),
            pltpu.VMEM((nb,), jnp.float32),
            pltpu.SemaphoreType.DMA,
        ],
        compiler_params=pltpu.CompilerParams(needs_layout_passes=False),
    )(functools.partial(_deg_body, nb, ci))
    return k(dst_r).reshape(NC * NS, nb)


_K = 3


def _agg_body(cc0, cc1, n_out, ht_hbm, srcf_hbm, dstf_hbm,
              zeros_hbm, out_hbm,
              s0, s1, s2, d0, d1, d2, r0, r1, r2, acc_sh,
              issem, idsem, gsem, ssem):
    sidx = (s0, s1, s2)
    didx = (d0, d1, d2)
    rows = (r0, r1, r2)
    c = lax.axis_index("c")
    s = lax.axis_index("s")
    base = jnp.where(c == 0, s * cc0, NS * cc0 + s * cc1)
    nck = jnp.where(c == 0, cc0, cc1)

    def ld_s(j, b):
        return pltpu.make_async_copy(
            srcf_hbm.at[pl.ds((base + j) * CH, CH)], sidx[b], issem.at[b])

    def ld_d(j, b):
        return pltpu.make_async_copy(
            dstf_hbm.at[pl.ds((base + j) * CH, CH)], didx[b], idsem.at[b])

    def gather(b):
        return pltpu.make_async_copy(
            ht_hbm.at[sidx[b]], rows[b], gsem.at[b])

    def scatter(b):
        return pltpu.make_async_copy(
            rows[b], acc_sh.at[didx[b]], ssem.at[b])

    rpt = n_out // NS
    pltpu.sync_copy(zeros_hbm.at[pl.ds(s * rpt, rpt)],
                    acc_sh.at[pl.ds(s * rpt, rpt)])
    plsc.subcore_barrier()

    for b in range(_K):
        ld_s(b, b).start()
        ld_d(b, b).start()
    for b in range(_K):
        ld_s(b, b).wait()
        gather(b).start()

    def group(g, _):
        j0 = g * _K
        for b in range(_K):
            j = j0 + b
            jn = j + _K
            gather(b).wait()

            @pl.when(jn < nck)
            def _():
                ld_s(jn, b).start()
            ld_d(j, b).wait()
            scatter(b).start(add=True)
            scatter(b).wait()

            @pl.when(jn < nck)
            def _():
                ld_d(jn, b).start()
                ld_s(jn, b).wait()
                gather(b).start()
        return _

    lax.fori_loop(0, nck // _K, group, None)
    plsc.subcore_barrier()
    pltpu.sync_copy(acc_sh.at[pl.ds(s * rpt, rpt)],
                    out_hbm.at[c, pl.ds(s * rpt, rpt)])


def _aggregate(ht, src_flat, dst_flat, zeros_nd, cc0, cc1):
    n, d = ht.shape
    n1 = zeros_nd.shape[0]
    idx_t = pltpu.VMEM((CH,), jnp.int32)
    row_t = pltpu.VMEM((CH, d), jnp.float32)
    k = functools.partial(
        pl.kernel,
        out_type=jax.ShapeDtypeStruct((NC, n1, d), jnp.float32),
        mesh=plsc.VectorSubcoreMesh(core_axis_name="c", subcore_axis_name="s",
                                    num_cores=NC, num_subcores=NS),
        scratch_types=[
            idx_t, idx_t, idx_t, idx_t, idx_t, idx_t,
            row_t, row_t, row_t,
            pltpu.VMEM_SHARED((n1, d), jnp.float32),
            pltpu.SemaphoreType.DMA((_K,)),
            pltpu.SemaphoreType.DMA((_K,)),
            pltpu.SemaphoreType.DMA((_K,)),
            pltpu.SemaphoreType.DMA((_K,)),
        ],
    )(functools.partial(_agg_body, cc0, cc1, n1))
    return k(ht, src_flat, dst_flat, zeros_nd)


def _dinv_kernel(hist_ref, out_ref):
    deg = jnp.sum(hist_ref[...], axis=0) + 1.0
    out_ref[...] = lax.rsqrt(deg)


def _compute_dinv(hist_parts, nb):
    rows = nb // 128
    h3 = hist_parts.reshape(NC * NS, rows, 128)
    out = pl.pallas_call(
        _dinv_kernel,
        out_shape=jax.ShapeDtypeStruct((rows, 128), jnp.float32),
    )(h3)
    return out.reshape(nb)


def _mm1_kernel(x_ref, w_ref, dinv_ref, out_ref):
    out_ref[...] = jnp.dot(x_ref[...], w_ref[...],
                           preferred_element_type=jnp.float32) * dinv_ref[...]


def _fused_kernel(p0_ref, p1_ref, htp_ref, dinv_ref, b_ref, w_ref, out_ref):
    h = (p0_ref[...] + p1_ref[...] + htp_ref[...]) * dinv_ref[...] + b_ref[...]
    h = jnp.maximum(h, 0.0)
    out_ref[...] = jnp.dot(h, w_ref[...],
                           preferred_element_type=jnp.float32) * dinv_ref[...]


def _final_kernel(dout, p0_ref, p1_ref, htp_ref, dinv_ref, b_ref, out_ref):
    s = (p0_ref[...] + p1_ref[...] + htp_ref[...])[:, :dout]
    out_ref[...] = s * dinv_ref[...] + b_ref[...]


_BN = 1000


def _row_spec(d):
    return pl.BlockSpec((_BN, d), lambda i: (i, 0))


def _full_spec(shape):
    return pl.BlockSpec(shape, lambda i: (0,) * len(shape))


def _mm1(x, w, dinv_col):
    n, din = x.shape
    dout = w.shape[1]
    return pl.pallas_call(
        _mm1_kernel,
        grid=(n // _BN,),
        in_specs=[_row_spec(din), _full_spec(w.shape),
                  pl.BlockSpec((_BN, 1), lambda i: (i, 0))],
        out_specs=_row_spec(dout),
        out_shape=jax.ShapeDtypeStruct((n, dout), jnp.float32),
    )(x, w, dinv_col)


def _fused(parts, htp, dinv_col, b, w):
    n, din = htp.shape
    dout = w.shape[1]
    return pl.pallas_call(
        _fused_kernel,
        grid=(n // _BN,),
        in_specs=[_row_spec(din), _row_spec(din), _row_spec(din),
                  pl.BlockSpec((_BN, 1), lambda i: (i, 0)),
                  _full_spec((1, din)), _full_spec(w.shape)],
        out_specs=_row_spec(dout),
        out_shape=jax.ShapeDtypeStruct((n, dout), jnp.float32),
    )(parts[0], parts[1], htp, dinv_col, b.reshape(1, din), w)


def _final(parts, htp, dinv_col, b):
    n, d = htp.shape
    dout = b.shape[0]
    return pl.pallas_call(
        functools.partial(_final_kernel, dout),
        grid=(n // _BN,),
        in_specs=[_row_spec(d), _row_spec(d), _row_spec(d),
                  pl.BlockSpec((_BN, 1), lambda i: (i, 0)),
                  _full_spec((1, dout))],
        out_specs=_row_spec(dout),
        out_shape=jax.ShapeDtypeStruct((n, dout), jnp.float32),
    )(parts[0], parts[1], htp, dinv_col, b.reshape(1, dout))


def kernel(x, edge_index, W1, b1, W2, b2, W3, b3):
    n, d_in = x.shape
    e = edge_index.shape[1]
    src = edge_index[0]
    dst = edge_index[1]

    nb = _cdiv(n + 1, 128) * 128
    n1 = _cdiv(n + 1, NS * 8) * NS * 8

    cc_tot = NC * _cdiv(e, NC * NS * CH * _K) * _K
    cc0 = (cc_tot // (2 * _K)) * _K + 69
    cc1 = cc_tot - cc0
    e_pad = NS * cc_tot * CH
    src_r = jnp.concatenate([src, jnp.zeros((e_pad - e,), jnp.int32)])
    dst_r = jnp.concatenate([dst, jnp.full((e_pad - e,), n, jnp.int32)])

    zeros_wide = jnp.zeros((n1, d_in), jnp.float32)

    hist_parts = _degree_hist(dst, n, nb)
    dinv = _compute_dinv(hist_parts, nb)
    dinv_col = dinv[:n].reshape(n, 1)

    ht1 = _mm1(x, W1, dinv_col)
    p1 = _aggregate(ht1, src_r, dst_r, zeros_wide, cc0, cc1)
    ht2 = _fused(p1, ht1, dinv_col, b1, W2)
    p2 = _aggregate(ht2, src_r, dst_r, zeros_wide, cc0, cc1)
    w3_pad = jnp.concatenate(
        [W3, jnp.zeros((W3.shape[0], 128 - W3.shape[1]), jnp.float32)], axis=1)
    ht3 = _fused(p2, ht2, dinv_col, b2, w3_pad)
    p3 = _aggregate(ht3, src_r, dst_r, zeros_wide, cc0, cc1)
    mu = _final(p3, ht3, dinv_col, b3)
    return (mu, mu)
